# baseline (speedup 1.0000x reference)
import functools

import jax
import jax.numpy as jnp
from jax import lax
from jax.experimental import pallas as pl
from jax.experimental.pallas import tpu as pltpu
from jax.experimental.pallas import tpu_sc as plsc

BATCH_SIZE = 16384
D0 = 64

_info = plsc.get_sparse_core_info()
_NC, _NS = _info.num_cores, _info.num_subcores
_NW = _NC * _NS
_BPW = BATCH_SIZE // _NW
_L = 16
_PHASE = 64
_NPHASE = _BPW // _PHASE
_CHUNK = 128
_NCHUNK = _BPW // _CHUNK


def _body0(d_hbm, idx_hbm, out_hbm, idx_v, stage_v, out_v, sem):
    wid = lax.axis_index("s") * _NC + lax.axis_index("c")
    pltpu.sync_copy(idx_hbm.at[wid], idx_v)
    copies = []
    for j in range(_NCHUNK):
        copies.append(pltpu.async_copy(
            d_hbm.at[idx_v.at[j]], stage_v.at[pl.ds(j * _CHUNK, _CHUNK)], sem))
    for c in copies:
        c.wait()

    def win(w):
        base = w * _L
        ivec = jax.lax.iota(jnp.int32, _L) + base
        for c in range(D0):
            cvec = jax.lax.iota(jnp.int32, _L) * 0 + c
            vals = plsc.load_gather(stage_v, [ivec, cvec])
            out_v[c, pl.ds(base, _L)] = vals
    pl.loop(0, _BPW // _L)(win)

    pltpu.sync_copy(out_v, out_hbm.at[:, pl.ds(wid * _BPW, _BPW)])


def _body1(d1_hbm, idx_hbm, out_hbm, idx_v, rows_v, sem):
    wid = lax.axis_index("s") * _NC + lax.axis_index("c")
    pltpu.sync_copy(idx_hbm.at[wid], idx_v)
    copies = []
    for j in range(_NCHUNK):
        sl = pl.ds(j * _CHUNK, _CHUNK)
        copies.append(pltpu.async_copy(
            d1_hbm.at[idx_v.at[j]], rows_v.at[sl], sem))
    for c in copies:
        c.wait()
    pltpu.sync_copy(rows_v, out_hbm.at[pl.ds(wid * _BPW, _BPW)])


@jax.jit
def _run(data0, d1flat, idx3):
    mesh = plsc.VectorSubcoreMesh(core_axis_name="c", subcore_axis_name="s")
    f0 = functools.partial(
        pl.kernel,
        mesh=mesh,
        out_type=jax.ShapeDtypeStruct((D0, BATCH_SIZE), jnp.float32),
        scratch_types=[
            pltpu.VMEM((_NCHUNK, _CHUNK), jnp.int32),
            pltpu.VMEM((_BPW, D0), jnp.float32),
            pltpu.VMEM((D0, _BPW), jnp.float32),
            pltpu.SemaphoreType.DMA,
        ],
        compiler_params=pltpu.CompilerParams(use_tc_tiling_on_sc=False,
                                             needs_layout_passes=False),
    )(_body0)
    f1 = functools.partial(
        pl.kernel,
        mesh=mesh,
        out_type=jax.ShapeDtypeStruct((BATCH_SIZE,), jnp.float32),
        scratch_types=[
            pltpu.VMEM((_NCHUNK, _CHUNK), jnp.int32),
            pltpu.VMEM((_BPW,), jnp.float32),
            pltpu.SemaphoreType.DMA,
        ],
        compiler_params=pltpu.CompilerParams(use_tc_tiling_on_sc=False),
    )(_body1)
    out0 = f0(data0, idx3)
    out1 = f1(d1flat, idx3)
    return out0, out1


def kernel(data0, data1, step):
    loader_key = jax.random.key(42)
    key = jax.random.fold_in(loader_key, step)
    idx = jax.random.randint(key, (BATCH_SIZE,), minval=0,
                             maxval=data0.shape[0], dtype=jnp.int32)
    idx3 = idx.reshape(_NW, _NCHUNK, _CHUNK)
    out0, out1 = _run(data0, data1.reshape(-1), idx3)
    return out0.T, out1.reshape(BATCH_SIZE, 1)


# speedup vs baseline: 1.4176x; 1.4176x over previous
import functools

import jax
import jax.numpy as jnp
from jax import lax
from jax.experimental import pallas as pl
from jax.experimental.pallas import tpu as pltpu
from jax.experimental.pallas import tpu_sc as plsc

BATCH_SIZE = 16384
D0 = 64

_info = plsc.get_sparse_core_info()
_NC, _NS = _info.num_cores, _info.num_subcores
_NW = _NC * _NS
_BPW = BATCH_SIZE // _NW
_L = 16
_PHASE = 64
_NPHASE = _BPW // _PHASE
_CHUNK = 128
_NCHUNK = _BPW // _CHUNK


def _body0(d_hbm, idx_hbm, out_hbm, idx_v, ring_v, out_v, sem):
    wid = lax.axis_index("s") * _NC + lax.axis_index("c")
    pltpu.sync_copy(idx_hbm.at[wid], idx_v)

    def extract_phase(p):
        def win(w):
            base = p * _PHASE + w * _L
            svec = idx_v[pl.ds(base, _L)] & 7
            slotvec = jax.lax.iota(jnp.int32, _L) + w * _L
            for c in range(D0):
                cvec = jax.lax.iota(jnp.int32, _L) * 0 + c
                vals = plsc.load_gather(ring_v, [slotvec, svec, cvec])
                out_v[c, pl.ds(base, _L)] = vals
        pl.loop(0, _PHASE // _L)(win)

    for p in range(_NPHASE):
        descs = []
        for w in range(_PHASE // _L):
            vec = idx_v[pl.ds(p * _PHASE + w * _L, _L)]
            for t in range(_L):
                r = vec[t]
                t8 = pl.multiple_of((r >> 3) * 8, 8)
                descs.append(pltpu.async_copy(
                    d_hbm.at[pl.ds(t8, 8), :], ring_v.at[w * _L + t], sem))
        for d in descs:
            d.wait()
        extract_phase(p)

    pltpu.sync_copy(out_v, out_hbm.at[:, pl.ds(wid * _BPW, _BPW)])


def _body1(d1_hbm, idx_hbm, out_hbm, idx_v, rows_v, sem):
    wid = lax.axis_index("s") * _NC + lax.axis_index("c")
    pltpu.sync_copy(idx_hbm.at[wid], idx_v)
    copies = []
    for j in range(_NCHUNK):
        sl = pl.ds(j * _CHUNK, _CHUNK)
        copies.append(pltpu.async_copy(
            d1_hbm.at[idx_v.at[j]], rows_v.at[sl], sem))
    for c in copies:
        c.wait()
    pltpu.sync_copy(rows_v, out_hbm.at[pl.ds(wid * _BPW, _BPW)])


@jax.jit
def _run(data0, d1flat, idx2, idx3):
    mesh = plsc.VectorSubcoreMesh(core_axis_name="c", subcore_axis_name="s")
    f0 = functools.partial(
        pl.kernel,
        mesh=mesh,
        out_type=jax.ShapeDtypeStruct((D0, BATCH_SIZE), jnp.float32),
        scratch_types=[
            pltpu.VMEM((_BPW,), jnp.int32),
            pltpu.VMEM((_PHASE, 8, D0), jnp.float32),
            pltpu.VMEM((D0, _BPW), jnp.float32),
            pltpu.SemaphoreType.DMA,
        ],
        compiler_params=pltpu.CompilerParams(needs_layout_passes=False),
    )(_body0)
    f1 = functools.partial(
        pl.kernel,
        mesh=mesh,
        out_type=jax.ShapeDtypeStruct((BATCH_SIZE,), jnp.float32),
        scratch_types=[
            pltpu.VMEM((_NCHUNK, _CHUNK), jnp.int32),
            pltpu.VMEM((_BPW,), jnp.float32),
            pltpu.SemaphoreType.DMA,
        ],
        compiler_params=pltpu.CompilerParams(use_tc_tiling_on_sc=False),
    )(_body1)
    out0 = f0(data0, idx2)
    out1 = f1(d1flat, idx3)
    return out0, out1


def kernel(data0, data1, step):
    loader_key = jax.random.key(42)
    key = jax.random.fold_in(loader_key, step)
    idx = jax.random.randint(key, (BATCH_SIZE,), minval=0,
                             maxval=data0.shape[0], dtype=jnp.int32)
    idx2 = idx.reshape(_NW, _BPW)
    idx3 = idx.reshape(_NW, _NCHUNK, _CHUNK)
    out0, out1 = _run(data0, data1.reshape(-1), idx2, idx3)
    return out0.T, out1.reshape(BATCH_SIZE, 1)


# speedup vs baseline: 1.4902x; 1.0512x over previous
import functools

import jax
import jax.numpy as jnp
from jax import lax
from jax.experimental import pallas as pl
from jax.experimental.pallas import tpu as pltpu
from jax.experimental.pallas import tpu_sc as plsc

BATCH_SIZE = 16384
D0 = 64

_info = plsc.get_sparse_core_info()
_NC, _NS = _info.num_cores, _info.num_subcores
_NW = _NC * _NS
_BPW = BATCH_SIZE // _NW
_L = 16
_PHASE = 32
_NPHASE = _BPW // _PHASE
_CHUNK = 128
_NCHUNK = _BPW // _CHUNK


def _body0(d_hbm, idx_hbm, out_hbm, idx_v, ring_v, out_v, sem):
    wid = lax.axis_index("s") * _NC + lax.axis_index("c")
    pltpu.sync_copy(idx_hbm.at[wid], idx_v)

    def extract_phase(p):
        half = (p % 2) * _PHASE
        def win(w):
            base = p * _PHASE + w * _L
            svec = idx_v[pl.ds(base, _L)] & 7
            slotvec = jax.lax.iota(jnp.int32, _L) + (half + w * _L)
            for c in range(D0):
                cvec = jax.lax.iota(jnp.int32, _L) * 0 + c
                vals = plsc.load_gather(ring_v, [slotvec, svec, cvec])
                out_v[c, pl.ds(base, _L)] = vals
        pl.loop(0, _PHASE // _L)(win)

    def fire_phase(p):
        descs = []
        half = (p % 2) * _PHASE
        for w in range(_PHASE // _L):
            vec = idx_v[pl.ds(p * _PHASE + w * _L, _L)]
            for t in range(_L):
                r = vec[t]
                t8 = pl.multiple_of((r >> 3) * 8, 8)
                descs.append(pltpu.async_copy(
                    d_hbm.at[pl.ds(t8, 8), :],
                    ring_v.at[half + w * _L + t], sem))
        return descs

    prev = fire_phase(0)
    for p in range(1, _NPHASE):
        cur = fire_phase(p)
        for d in prev:
            d.wait()
        extract_phase(p - 1)
        prev = cur
    for d in prev:
        d.wait()
    extract_phase(_NPHASE - 1)

    pltpu.sync_copy(out_v, out_hbm.at[:, pl.ds(wid * _BPW, _BPW)])


def _body1(d1_hbm, idx_hbm, out_hbm, idx_v, rows_v, sem):
    wid = lax.axis_index("s") * _NC + lax.axis_index("c")
    pltpu.sync_copy(idx_hbm.at[wid], idx_v)
    copies = []
    for j in range(_NCHUNK):
        sl = pl.ds(j * _CHUNK, _CHUNK)
        copies.append(pltpu.async_copy(
            d1_hbm.at[idx_v.at[j]], rows_v.at[sl], sem))
    for c in copies:
        c.wait()
    pltpu.sync_copy(rows_v, out_hbm.at[pl.ds(wid * _BPW, _BPW)])


@jax.jit
def _run(data0, d1flat, idx2, idx3):
    mesh = plsc.VectorSubcoreMesh(core_axis_name="c", subcore_axis_name="s")
    f0 = functools.partial(
        pl.kernel,
        mesh=mesh,
        out_type=jax.ShapeDtypeStruct((D0, BATCH_SIZE), jnp.float32),
        scratch_types=[
            pltpu.VMEM((_BPW,), jnp.int32),
            pltpu.VMEM((2 * _PHASE, 8, D0), jnp.float32),
            pltpu.VMEM((D0, _BPW), jnp.float32),
            pltpu.SemaphoreType.DMA,
        ],
        compiler_params=pltpu.CompilerParams(needs_layout_passes=False),
    )(_body0)
    f1 = functools.partial(
        pl.kernel,
        mesh=mesh,
        out_type=jax.ShapeDtypeStruct((BATCH_SIZE,), jnp.float32),
        scratch_types=[
            pltpu.VMEM((_NCHUNK, _CHUNK), jnp.int32),
            pltpu.VMEM((_BPW,), jnp.float32),
            pltpu.SemaphoreType.DMA,
        ],
        compiler_params=pltpu.CompilerParams(use_tc_tiling_on_sc=False),
    )(_body1)
    out0 = f0(data0, idx2)
    out1 = f1(d1flat, idx3)
    return out0, out1


def kernel(data0, data1, step):
    loader_key = jax.random.key(42)
    key = jax.random.fold_in(loader_key, step)
    idx = jax.random.randint(key, (BATCH_SIZE,), minval=0,
                             maxval=data0.shape[0], dtype=jnp.int32)
    idx2 = idx.reshape(_NW, _BPW)
    idx3 = idx.reshape(_NW, _NCHUNK, _CHUNK)
    out0, out1 = _run(data0, data1.reshape(-1), idx2, idx3)
    return out0.T, out1.reshape(BATCH_SIZE, 1)
